# baseline (speedup 1.0000x reference)
import functools

import jax
import jax.numpy as jnp
from jax import lax
from jax.experimental import pallas as pl
from jax.experimental.pallas import tpu as pltpu
from jax.experimental.pallas import tpu_sc as plsc

B = 32
L_SEQ = 4096
L_TREE = L_SEQ - 1
DEPTH = 12
D_FEAT = 128
LANES = 16
NGRP = D_FEAT // LANES
N_LEAVES = 2 ** (DEPTH - 1)
CHUNK = 128
NCHUNK = N_LEAVES // CHUNK
NPAIR = NCHUNK // 2

_INTL_OFF = {10: 0, 9: 64, 8: 96, 7: 112, 6: 120, 5: 124}
_INTL_ROWS = 126
_NLVL = 6
_TOP_ROWS = 31


def _tanh(t):
    return 1.0 - 2.0 / (1.0 + jnp.exp(t + t))


def _pair_reduce(src_ref, src_pre, src_base, dst_ref, dst_pre, dst_base,
                 n_out, unroll=1):
    if n_out % unroll:
        unroll = 1

    def body(i, carry):
        for u in range(unroll):
            j = i * unroll + u
            for k in range(NGRP):
                sl = pl.ds(LANES * k, LANES)
                a = src_ref[(*src_pre, src_base + 2 * j, sl)]
                b = src_ref[(*src_pre, src_base + 2 * j + 1, sl)]
                dst_ref[(*dst_pre, dst_base + j, sl)] = _tanh(a + b)
        return carry

    lax.fori_loop(0, n_out // unroll, body, 0)


def _chunk_levels(c):
    out = []
    for lvl in range(10, 4, -1):
        cnt = 2 ** (lvl - 4)
        out.append((_INTL_OFF[lvl], cnt, (2 ** lvl - 1) + c * cnt))
    return out


def _sc_body(seq_hbm, out_hbm, inbuf, intl, top, tail,
             rsem, tsem, psem0, psem1, lsem0, lsem1):
    wid = lax.axis_index("s") * 2 + lax.axis_index("c")
    b = wid
    psem = (psem0, psem1)
    lsem = (lsem0, lsem1)

    def leaf_slice(c):
        return seq_hbm.at[b, pl.ds((N_LEAVES - 1) + c * CHUNK, CHUNK)]

    def pass_slice(c):
        return out_hbm.at[b, pl.ds((N_LEAVES - 1) + c * CHUNK, CHUNK)]

    def wait_levels(c, s):
        for off, cnt, base0 in _chunk_levels(0):
            pltpu.make_async_copy(
                intl.at[s, pl.ds(off, cnt)],
                out_hbm.at[b, pl.ds(base0 + c * cnt, cnt)], lsem[s]).wait()

    def compute_chunk(c, s):
        _pair_reduce(inbuf, (s,), 0, intl, (s,), _INTL_OFF[10], 64, unroll=2)
        _pair_reduce(intl, (s,), _INTL_OFF[10], intl, (s,), _INTL_OFF[9], 32,
                     unroll=2)
        _pair_reduce(intl, (s,), _INTL_OFF[9], intl, (s,), _INTL_OFF[8], 16,
                     unroll=2)
        _pair_reduce(intl, (s,), _INTL_OFF[8], intl, (s,), _INTL_OFF[7], 8)
        _pair_reduce(intl, (s,), _INTL_OFF[7], intl, (s,), _INTL_OFF[6], 4)
        _pair_reduce(intl, (s,), _INTL_OFF[6], intl, (s,), _INTL_OFF[5], 2)
        _pair_reduce(intl, (s,), _INTL_OFF[5], top, (), 15 + c, 1)

    def issue_writes(c, s):
        pltpu.async_copy(inbuf.at[s], pass_slice(c), psem[s])
        for off, cnt, hbm_base in _chunk_levels(c):
            pltpu.async_copy(intl.at[s, pl.ds(off, cnt)],
                             out_hbm.at[b, pl.ds(hbm_base, cnt)], lsem[s])

    pltpu.async_copy(leaf_slice(0), inbuf.at[0], rsem)
    pltpu.async_copy(seq_hbm.at[b, pl.ds(L_SEQ - 1, 1)], tail, tsem)

    def pair_step(t, carry):
        c0 = 2 * t
        c1 = 2 * t + 1

        pltpu.make_async_copy(leaf_slice(c0), inbuf.at[0], rsem).wait()

        @pl.when(t >= 1)
        def _slot1_pass_done():
            pltpu.make_async_copy(inbuf.at[1], pass_slice(c0 - 1),
                                  psem[1]).wait()

        pltpu.async_copy(leaf_slice(c1), inbuf.at[1], rsem)

        @pl.when(t >= 1)
        def _slot0_levels_done():
            wait_levels(c0 - 2, 0)

        compute_chunk(c0, 0)
        issue_writes(c0, 0)

        pltpu.make_async_copy(leaf_slice(c1), inbuf.at[1], rsem).wait()

        @pl.when(t < NPAIR - 1)
        def _prefetch_next():
            pltpu.make_async_copy(inbuf.at[0], pass_slice(c0), psem[0]).wait()
            pltpu.async_copy(leaf_slice(c1 + 1), inbuf.at[0], rsem)

        @pl.when(t >= 1)
        def _slot1_levels_done():
            wait_levels(c1 - 2, 1)

        compute_chunk(c1, 1)
        issue_writes(c1, 1)
        return carry

    lax.fori_loop(0, NPAIR, pair_step, 0)

    pltpu.make_async_copy(inbuf.at[0], pass_slice(NCHUNK - 2), psem[0]).wait()
    pltpu.make_async_copy(inbuf.at[1], pass_slice(NCHUNK - 1), psem[1]).wait()
    wait_levels(NCHUNK - 2, 0)
    wait_levels(NCHUNK - 1, 1)

    _pair_reduce(top, (), 15, top, (), 7, 8)
    _pair_reduce(top, (), 7, top, (), 3, 4)
    _pair_reduce(top, (), 3, top, (), 1, 2)
    _pair_reduce(top, (), 1, top, (), 0, 1)
    pltpu.sync_copy(top, out_hbm.at[b, pl.ds(0, _TOP_ROWS)])

    pltpu.make_async_copy(seq_hbm.at[b, pl.ds(L_SEQ - 1, 1)], tail,
                          tsem).wait()
    pltpu.sync_copy(tail, out_hbm.at[b, pl.ds(L_SEQ - 1, 1)])


def kernel(seqs, parent_idx, node_level):
    del parent_idx, node_level
    mesh = plsc.VectorSubcoreMesh(core_axis_name="c", subcore_axis_name="s")
    run = functools.partial(
        pl.kernel,
        out_type=jax.ShapeDtypeStruct((B, L_SEQ, D_FEAT), jnp.float32),
        mesh=mesh,
        scratch_types=[
            pltpu.VMEM((2, CHUNK, D_FEAT), jnp.float32),
            pltpu.VMEM((2, _INTL_ROWS, D_FEAT), jnp.float32),
            pltpu.VMEM((_TOP_ROWS, D_FEAT), jnp.float32),
            pltpu.VMEM((1, D_FEAT), jnp.float32),
            pltpu.SemaphoreType.DMA,
            pltpu.SemaphoreType.DMA,
            pltpu.SemaphoreType.DMA,
            pltpu.SemaphoreType.DMA,
            pltpu.SemaphoreType.DMA,
            pltpu.SemaphoreType.DMA,
        ],
        compiler_params=pltpu.CompilerParams(use_tc_tiling_on_sc=False),
    )(_sc_body)
    return run(seqs)


# speedup vs baseline: 1.0055x; 1.0055x over previous
import functools

import jax
import jax.numpy as jnp
from jax import lax
from jax.experimental import pallas as pl
from jax.experimental.pallas import tpu as pltpu
from jax.experimental.pallas import tpu_sc as plsc

B = 32
L_SEQ = 4096
L_TREE = L_SEQ - 1
DEPTH = 12
D_FEAT = 128
LANES = 16
NGRP = D_FEAT // LANES
N_LEAVES = 2 ** (DEPTH - 1)
CHUNK = 128
NCHUNK = N_LEAVES // CHUNK
NPAIR = NCHUNK // 2

_LVL_BUF = {10: ("a", 0), 9: ("b", 0), 8: ("a", 64), 7: ("b", 32),
            6: ("a", 80), 5: ("b", 40)}
_BUFA_ROWS = 84
_BUFB_ROWS = 42
_NLVL = 6
_TOP_ROWS = 31


def _tanh(t):
    return 1.0 - 2.0 / (1.0 + jnp.exp(t + t))


def _pair_reduce(src_ref, src_pre, src_base, dst_ref, dst_pre, dst_base,
                 n_out, unroll=1):
    if n_out % unroll:
        unroll = 1

    def body(i, carry):
        for u in range(unroll):
            j = i * unroll + u
            for k in range(NGRP):
                sl = pl.ds(LANES * k, LANES)
                a = src_ref[(*src_pre, src_base + 2 * j, sl)]
                b = src_ref[(*src_pre, src_base + 2 * j + 1, sl)]
                dst_ref[(*dst_pre, dst_base + j, sl)] = _tanh(a + b)
        return carry

    lax.fori_loop(0, n_out // unroll, body, 0)


def _chunk_levels(c):
    out = []
    for lvl in range(10, 4, -1):
        key, off = _LVL_BUF[lvl]
        cnt = 2 ** (lvl - 4)
        out.append((key, off, cnt, (2 ** lvl - 1) + c * cnt))
    return out


def _sc_body(seq_hbm, out_hbm, inbuf, bufa, bufb, top, tail,
             rsem, tsem, psem0, psem1, lsem0, lsem1):
    wid = lax.axis_index("s") * 2 + lax.axis_index("c")
    b = wid
    psem = (psem0, psem1)
    lsem = (lsem0, lsem1)

    def leaf_slice(c):
        return seq_hbm.at[b, pl.ds((N_LEAVES - 1) + c * CHUNK, CHUNK)]

    def pass_slice(c):
        return out_hbm.at[b, pl.ds((N_LEAVES - 1) + c * CHUNK, CHUNK)]

    lvlbuf = {"a": bufa, "b": bufb}

    def wait_levels(c, s):
        for key, off, cnt, base0 in _chunk_levels(0):
            pltpu.make_async_copy(
                lvlbuf[key].at[s, pl.ds(off, cnt)],
                out_hbm.at[b, pl.ds(base0 + c * cnt, cnt)], lsem[s]).wait()

    def compute_chunk(c, s):
        _pair_reduce(inbuf, (s,), 0, bufa, (s,), 0, 64, unroll=2)
        _pair_reduce(bufa, (s,), 0, bufb, (s,), 0, 32, unroll=2)
        _pair_reduce(bufb, (s,), 0, bufa, (s,), 64, 16, unroll=2)
        _pair_reduce(bufa, (s,), 64, bufb, (s,), 32, 8, unroll=2)
        _pair_reduce(bufb, (s,), 32, bufa, (s,), 80, 4, unroll=2)
        _pair_reduce(bufa, (s,), 80, bufb, (s,), 40, 2, unroll=2)
        _pair_reduce(bufb, (s,), 40, top, (), 15 + c, 1)

    def issue_writes(c, s):
        pltpu.async_copy(inbuf.at[s], pass_slice(c), psem[s])
        for key, off, cnt, hbm_base in _chunk_levels(c):
            pltpu.async_copy(lvlbuf[key].at[s, pl.ds(off, cnt)],
                             out_hbm.at[b, pl.ds(hbm_base, cnt)], lsem[s])

    pltpu.async_copy(leaf_slice(0), inbuf.at[0], rsem)
    pltpu.async_copy(seq_hbm.at[b, pl.ds(L_SEQ - 1, 1)], tail, tsem)

    def pair_step(t, carry):
        c0 = 2 * t
        c1 = 2 * t + 1

        pltpu.make_async_copy(leaf_slice(c0), inbuf.at[0], rsem).wait()

        @pl.when(t >= 1)
        def _slot1_pass_done():
            pltpu.make_async_copy(inbuf.at[1], pass_slice(c0 - 1),
                                  psem[1]).wait()

        pltpu.async_copy(leaf_slice(c1), inbuf.at[1], rsem)

        @pl.when(t >= 1)
        def _slot0_levels_done():
            wait_levels(c0 - 2, 0)

        compute_chunk(c0, 0)
        issue_writes(c0, 0)

        pltpu.make_async_copy(leaf_slice(c1), inbuf.at[1], rsem).wait()

        @pl.when(t < NPAIR - 1)
        def _prefetch_next():
            pltpu.make_async_copy(inbuf.at[0], pass_slice(c0), psem[0]).wait()
            pltpu.async_copy(leaf_slice(c1 + 1), inbuf.at[0], rsem)

        @pl.when(t >= 1)
        def _slot1_levels_done():
            wait_levels(c1 - 2, 1)

        compute_chunk(c1, 1)
        issue_writes(c1, 1)
        return carry

    lax.fori_loop(0, NPAIR, pair_step, 0)

    pltpu.make_async_copy(inbuf.at[0], pass_slice(NCHUNK - 2), psem[0]).wait()
    pltpu.make_async_copy(inbuf.at[1], pass_slice(NCHUNK - 1), psem[1]).wait()
    wait_levels(NCHUNK - 2, 0)
    wait_levels(NCHUNK - 1, 1)

    _pair_reduce(top, (), 15, top, (), 7, 8)
    _pair_reduce(top, (), 7, top, (), 3, 4)
    _pair_reduce(top, (), 3, top, (), 1, 2)
    _pair_reduce(top, (), 1, top, (), 0, 1)
    pltpu.sync_copy(top, out_hbm.at[b, pl.ds(0, _TOP_ROWS)])

    pltpu.make_async_copy(seq_hbm.at[b, pl.ds(L_SEQ - 1, 1)], tail,
                          tsem).wait()
    pltpu.sync_copy(tail, out_hbm.at[b, pl.ds(L_SEQ - 1, 1)])


def kernel(seqs, parent_idx, node_level):
    del parent_idx, node_level
    mesh = plsc.VectorSubcoreMesh(core_axis_name="c", subcore_axis_name="s")
    run = functools.partial(
        pl.kernel,
        out_type=jax.ShapeDtypeStruct((B, L_SEQ, D_FEAT), jnp.float32),
        mesh=mesh,
        scratch_types=[
            pltpu.VMEM((2, CHUNK, D_FEAT), jnp.float32),
            pltpu.VMEM((2, _BUFA_ROWS, D_FEAT), jnp.float32),
            pltpu.VMEM((2, _BUFB_ROWS, D_FEAT), jnp.float32),
            pltpu.VMEM((_TOP_ROWS, D_FEAT), jnp.float32),
            pltpu.VMEM((1, D_FEAT), jnp.float32),
            pltpu.SemaphoreType.DMA,
            pltpu.SemaphoreType.DMA,
            pltpu.SemaphoreType.DMA,
            pltpu.SemaphoreType.DMA,
            pltpu.SemaphoreType.DMA,
            pltpu.SemaphoreType.DMA,
        ],
        compiler_params=pltpu.CompilerParams(use_tc_tiling_on_sc=False),
    )(_sc_body)
    return run(seqs)
